# baseline (speedup 1.0000x reference)
import functools

import jax
import jax.numpy as jnp
from jax.experimental import pallas as pl
from jax.experimental.pallas import tpu as pltpu

HIDDEN = 4096
NUM_CLASSES = 64
TOPK = 8
TOKENS = 8192

BT = 1024
KC = 4
HK = HIDDEN // KC


def _router_block(x_ref, w_ref, b_ref, o_ref, acc_ref):
    k = pl.program_id(1)
    partial = jax.lax.dot_general(
        x_ref[...], w_ref[...],
        (((1,), (1,)), ((), ())),
        preferred_element_type=jnp.float32,
    )

    @pl.when(k == 0)
    def _():
        acc_ref[...] = partial + b_ref[...]

    @pl.when(k != 0)
    def _():
        acc_ref[...] += partial

    @pl.when(k == KC - 1)
    def _():
        preds = acc_ref[...]
        rowmax = jnp.max(preds, axis=-1, keepdims=True)
        e = jnp.exp(preds - rowmax)
        denom = jnp.sum(e, axis=-1, keepdims=True)

        idx = jax.lax.broadcasted_iota(jnp.int32, preds.shape, 1)
        raw = jax.lax.bitcast_convert_type(preds, jnp.int32)
        ordered = jnp.where(raw < 0, raw ^ jnp.int32(0x7FFFFFFF), raw)
        ordered = (ordered & jnp.int32(~0x3F)) | (jnp.int32(63) - idx)
        kraw = jnp.where(ordered < 0, ordered ^ jnp.int32(0x7FFFFFFF), ordered)
        key = jax.lax.bitcast_convert_type(kraw, jnp.float32)

        keep = jnp.zeros(preds.shape, dtype=jnp.bool_)
        for _ in range(TOPK):
            m = jnp.max(key, axis=-1, keepdims=True)
            sel = key == m
            keep = jnp.logical_or(keep, sel)
            key = jnp.where(sel, -jnp.inf, key)

        o_ref[...] = jnp.where(keep, e / denom, 0.0)


@jax.jit
def kernel(x, W, b):
    b2 = b.reshape(1, NUM_CLASSES)
    grid = (TOKENS // BT, KC)
    return pl.pallas_call(
        _router_block,
        grid=grid,
        in_specs=[
            pl.BlockSpec((BT, HK), lambda i, k: (i, k)),
            pl.BlockSpec((NUM_CLASSES, HK), lambda i, k: (0, k)),
            pl.BlockSpec((1, NUM_CLASSES), lambda i, k: (0, 0)),
        ],
        out_specs=pl.BlockSpec((BT, NUM_CLASSES), lambda i, k: (i, 0)),
        out_shape=jax.ShapeDtypeStruct((TOKENS, NUM_CLASSES), jnp.float32),
        scratch_shapes=[pltpu.VMEM((BT, NUM_CLASSES), jnp.float32)],
    )(x, W, b2)


# speedup vs baseline: 1.4128x; 1.4128x over previous
import functools

import jax
import jax.numpy as jnp
from jax.experimental import pallas as pl

HIDDEN = 4096
NUM_CLASSES = 64
TOPK = 8
TOKENS = 8192

BT = 1024


def _router_block(x_ref, w_ref, b_ref, o_ref):
    preds = jax.lax.dot_general(
        x_ref[...], w_ref[...],
        (((1,), (1,)), ((), ())),
        preferred_element_type=jnp.float32,
    )
    preds = preds + b_ref[...]

    rowmax = jnp.max(preds, axis=-1, keepdims=True)
    e = jnp.exp(preds - rowmax)
    denom = jnp.sum(e, axis=-1, keepdims=True)

    idx = jax.lax.broadcasted_iota(jnp.int32, preds.shape, 1)
    raw = jax.lax.bitcast_convert_type(preds, jnp.int32)
    ordered = jnp.where(raw < 0, raw ^ jnp.int32(0x7FFFFFFF), raw)
    ordered = (ordered & jnp.int32(~0x3F)) | (jnp.int32(63) - idx)
    kraw = jnp.where(ordered < 0, ordered ^ jnp.int32(0x7FFFFFFF), ordered)
    key = jax.lax.bitcast_convert_type(kraw, jnp.float32)

    keep = jnp.zeros(preds.shape, dtype=jnp.bool_)
    for _ in range(TOPK):
        m = jnp.max(key, axis=-1, keepdims=True)
        sel = key == m
        keep = jnp.logical_or(keep, sel)
        key = jnp.where(sel, -jnp.inf, key)

    o_ref[...] = jnp.where(keep, e / denom, 0.0)


@jax.jit
def kernel(x, W, b):
    b2 = b.reshape(1, NUM_CLASSES)
    grid = (TOKENS // BT,)
    return pl.pallas_call(
        _router_block,
        grid=grid,
        in_specs=[
            pl.BlockSpec((BT, HIDDEN), lambda i: (i, 0)),
            pl.BlockSpec((NUM_CLASSES, HIDDEN), lambda i: (0, 0)),
            pl.BlockSpec((1, NUM_CLASSES), lambda i: (0, 0)),
        ],
        out_specs=pl.BlockSpec((BT, NUM_CLASSES), lambda i: (i, 0)),
        out_shape=jax.ShapeDtypeStruct((TOKENS, NUM_CLASSES), jnp.float32),
    )(x, W, b2)
